# baseline (speedup 1.0000x reference)
import functools

import jax
import jax.numpy as jnp
from jax import lax
from jax.experimental import pallas as pl
from jax.experimental.pallas import tpu as pltpu
from jax.experimental.pallas import tpu_sc as plsc

N = 50000
E = 200000
D = 128
OUT = 64
NC, NS = 2, 16
BR = 6400
ACC_R = BR + 64
DUMMY = BR
NBLK = 8
NPAD = NBLK * BR
PT = 12544
EPAD = NS * PT
CH = 3136
NCHUNK = PT // CH
VPC = CH // 16
BATCH = 48
SELCAP = CH + 96
ZR = ACC_R // NS
ZS = 101
WR = BR // NS
DGR = BR // 128
DGA = 56


def _seg_body(with_deg, *refs):
    if with_deg:
        (table, srcp, dstp, ident, s_out, deg_out, acc, dgacc, dst_c, src_c,
         sel_src, sel_dst, idsb0, idxb0, colb0, rowb0, idsb1, idxb1, colb1,
         rowb1, gbuf0, gbuf1, obuf0, obuf1, zrow,
         gfs0, gfs1, gds0, gds1, sfs0, sfs1, sds0, sds1) = refs
        idsb = (idsb0, idsb1)
        idxb = (idxb0, idxb1)
        colb = (colb0, colb1)
        rowb = (rowb0, rowb1)
        gbuf = (gbuf0, gbuf1)
        obuf = (obuf0, obuf1)
        gfs = (gfs0, gfs1)
        gds = (gds0, gds1)
        sfs = (sfs0, sfs1)
        sds = (sds0, sds1)
    else:
        (table, srcp, dstp, s_out, acc, dst_c, src_c,
         sel_src, sel_dst, idsb0, idxb0, idsb1, idxb1,
         gbuf0, gbuf1, zrow, gfs0, gfs1, sfs0, sfs1) = refs
        idsb = (idsb0, idsb1)
        idxb = (idxb0, idxb1)
        gbuf = (gbuf0, gbuf1)
        gfs = (gfs0, gfs1)
        sfs = (sfs0, sfs1)
    c = lax.axis_index("c")
    s = lax.axis_index("s")
    lanes = lax.iota(jnp.int32, 16)
    zf = jnp.zeros((16,), jnp.float32)

    def fill_zrow(i, carry):
        for k in range(D // 16):
            zrow[i, pl.ds(k * 16, 16)] = zf
        return carry
    lax.fori_loop(0, ZS, fill_zrow, 0)

    ebase = s * PT

    def phase_body(p):
        blk = p * NC + c
        lo = blk * BR

        for k in range(ZR // ZS):
            pltpu.sync_copy(zrow, acc.at[pl.ds(s * ZR + k * ZS, ZS)])
        if with_deg:
            @pl.when(s == 0)
            def _():
                pltpu.sync_copy(zrow.at[pl.ds(0, DGA)], dgacc)
        plsc.subcore_barrier()

        def chunk_body(ci, carry):
            cpd = pltpu.async_copy(dstp.at[pl.ds(ebase + ci * CH, CH)],
                                   dst_c, gfs[0])
            cps = pltpu.async_copy(srcp.at[pl.ds(ebase + ci * CH, CH)],
                                   src_c, gfs[1])
            cpd.wait()
            cps.wait()

            def vbody(j, cur):
                dv = dst_c[pl.ds(j * 16, 16)]
                sv = src_c[pl.ds(j * 16, 16)]
                ld = dv - lo
                inbi = (1 - lax.shift_right_logical(ld, 31)) * (
                    1 - lax.shift_right_logical(BR - 1 - ld, 31))
                outi = 1 - inbi
                pos = plsc.cumsum(inbi)
                tgt = inbi * (cur + pos - 1) + outi * (SELCAP - 16 + lanes)
                plsc.store_scatter(sel_src, [tgt], sv)
                plsc.store_scatter(sel_dst, [tgt], ld * inbi + DUMMY * outi)
                return cur + pos[15]
            nsel = lax.fori_loop(0, VPC, vbody, jnp.int32(0))

            tb = (nsel // 16) * 16
            for k in range(BATCH // 16 + 1):
                off = tb + k * 16
                m = (off + lanes) >= nsel
                olds = sel_src[pl.ds(off, 16)]
                oldd = sel_dst[pl.ds(off, 16)]
                sel_src[pl.ds(off, 16)] = jnp.where(m, 0, olds)
                sel_dst[pl.ds(off, 16)] = jnp.where(m, DUMMY, oldd)

            nb = (nsel + BATCH - 1) // BATCH

            def pbody(q, carry2):
                for io in range(2):
                    i = q * 2 + io
                    st = io
                    ot = 1 - io

                    @pl.when((i >= 2) & (i <= nb))
                    def _(st=st, i=i):
                        pltpu.make_async_copy(gbuf[st], acc.at[idxb[st]],
                                              sfs[st]).wait()
                        if with_deg:
                            pltpu.make_async_copy(obuf[st],
                                                  dgacc.at[rowb[st]],
                                                  sds[st]).wait()

                    @pl.when(i < nb)
                    def _(st=st, i=i):
                        for k in range(BATCH // 16):
                            sv_ = sel_src[pl.ds(i * BATCH + k * 16, 16)]
                            dv_ = sel_dst[pl.ds(i * BATCH + k * 16, 16)]
                            idsb[st][pl.ds(k * 16, 16)] = sv_
                            idxb[st][pl.ds(k * 16, 16)] = dv_
                            if with_deg:
                                colb[st][pl.ds(k * 16, 16)] = dv_ & 127
                                rowb[st][pl.ds(k * 16, 16)] = (
                                    lax.shift_right_logical(dv_, 7))
                        pltpu.async_copy(table.at[idsb[st]], gbuf[st],
                                         gfs[st])
                        if with_deg:
                            pltpu.async_copy(ident.at[colb[st]], obuf[st],
                                             gds[st])

                    @pl.when((i >= 1) & (i <= nb))
                    def _(ot=ot, i=i):
                        pltpu.make_async_copy(table.at[idsb[ot]], gbuf[ot],
                                              gfs[ot]).wait()
                        pltpu.async_copy(gbuf[ot], acc.at[idxb[ot]],
                                         sfs[ot], add=True)
                        if with_deg:
                            pltpu.make_async_copy(ident.at[colb[ot]],
                                                  obuf[ot], gds[ot]).wait()
                            pltpu.async_copy(obuf[ot], dgacc.at[rowb[ot]],
                                             sds[ot], add=True)
                return carry2
            lax.fori_loop(0, (nb + 2) // 2, pbody, 0)

            for par in range(2):
                @pl.when((nb >= 1) & (nb % 2 == (1 - par)))
                def _(par=par):
                    pltpu.make_async_copy(gbuf[par], acc.at[idxb[par]],
                                          sfs[par]).wait()
                    if with_deg:
                        pltpu.make_async_copy(obuf[par],
                                              dgacc.at[rowb[par]],
                                              sds[par]).wait()
            return carry
        lax.fori_loop(0, NCHUNK, chunk_body, 0)
        plsc.subcore_barrier()

        pltpu.sync_copy(acc.at[pl.ds(s * WR, WR)],
                        s_out.at[pl.ds(blk * BR + s * WR, WR)])
        if with_deg:
            @pl.when(s == 0)
            def _():
                pltpu.sync_copy(dgacc, deg_out.at[pl.ds(blk * DGA, DGA)])
        plsc.subcore_barrier()

    for p in range(NBLK // NC):
        phase_body(p)


@functools.lru_cache(maxsize=None)
def _make_seg(with_deg):
    out_type = [jax.ShapeDtypeStruct((NPAD, D), jnp.float32)]
    if with_deg:
        out_type.append(jax.ShapeDtypeStruct((NBLK * DGA, D), jnp.float32))
    scratch = [pltpu.VMEM_SHARED((ACC_R, D), jnp.float32)]
    if with_deg:
        scratch.append(pltpu.VMEM_SHARED((DGA, D), jnp.float32))
    scratch += [
        pltpu.VMEM((CH,), jnp.int32),
        pltpu.VMEM((CH,), jnp.int32),
        pltpu.VMEM((SELCAP,), jnp.int32),
        pltpu.VMEM((SELCAP,), jnp.int32),
    ]
    nidx = 4 if with_deg else 2
    for _ in range(2 * nidx):
        scratch.append(pltpu.VMEM((BATCH,), jnp.int32))
    scratch += [
        pltpu.VMEM((BATCH, D), jnp.float32),
        pltpu.VMEM((BATCH, D), jnp.float32),
    ]
    if with_deg:
        scratch += [
            pltpu.VMEM((BATCH, D), jnp.float32),
            pltpu.VMEM((BATCH, D), jnp.float32),
        ]
    scratch.append(pltpu.VMEM((ZS, D), jnp.float32))
    nsem = 8 if with_deg else 4
    for _ in range(nsem):
        scratch.append(pltpu.SemaphoreType.DMA)
    mesh = plsc.VectorSubcoreMesh(core_axis_name="c", subcore_axis_name="s",
                                  num_cores=NC, num_subcores=NS)
    return pl.kernel(functools.partial(_seg_body, with_deg),
                     out_type=tuple(out_type) if with_deg else out_type[0],
                     mesh=mesh,
                     scratch_types=scratch,
                     compiler_params=pltpu.CompilerParams(
                         needs_layout_passes=False),
                     name="seg_sum_deg" if with_deg else "seg_sum")


def _seg_deg(table, src, dst, ident):
    return _make_seg(True)(table, src, dst, ident)


def _seg(table, src, dst):
    return _make_seg(False)(table, src, dst)


def _dense(S_list, inv_list, X, W_list, Lw, relu, out_dim):
    nrel = len(S_list)
    BLK = 1000

    def body(*refs):
        Ss = refs[:nrel]
        invs = refs[nrel:2 * nrel]
        Xr = refs[2 * nrel]
        Ws = refs[2 * nrel + 1:3 * nrel + 1]
        Lr = refs[3 * nrel + 1]
        o = refs[-1]
        acc = jnp.dot(Xr[...], Lr[...], preferred_element_type=jnp.float32)
        for Sr, ir, Wr in zip(Ss, invs, Ws):
            acc = acc + jnp.dot(Sr[...], Wr[...],
                                preferred_element_type=jnp.float32) * ir[:, 0:1]
        o[...] = jnp.maximum(acc, 0.0) if relu else acc

    in_specs = (
        [pl.BlockSpec((BLK, D), lambda i: (i, 0))] * nrel
        + [pl.BlockSpec((BLK, 8), lambda i: (i, 0))] * nrel
        + [pl.BlockSpec((BLK, D), lambda i: (i, 0))]
        + [pl.BlockSpec((D, out_dim), lambda i: (0, 0))] * (nrel + 1)
    )
    return pl.pallas_call(
        body,
        grid=(N // BLK,),
        in_specs=in_specs,
        out_specs=pl.BlockSpec((BLK, out_dim), lambda i: (i, 0)),
        out_shape=jax.ShapeDtypeStruct((N, out_dim), jnp.float32),
    )(*S_list, *inv_list, X, *W_list, Lw)


def _pad_edges(ei):
    src = jnp.pad(ei[0], (0, EPAD - E))
    dst = jnp.pad(ei[1], (0, EPAD - E), constant_values=-1)
    return src, dst


def _inv_deg(deg_out):
    deg = deg_out.reshape(NBLK, DGA, D)[:, :DGR, :].reshape(NBLK * BR)[:N]
    inv = 1.0 / jnp.maximum(deg, 1.0)
    return jnp.broadcast_to(inv[:, None], (N, 8))


@jax.jit
def _impl(feat_paper, edge_index_writes, edge_index_cites,
          edge_index_written_by, embed_author, W1_writes, W1_cites,
          W1_written_by, L1_paper, L1_author, W2_writes, W2_cites,
          W2_written_by, L2_paper, L2_author):
    srcw, dstw = _pad_edges(edge_index_writes)
    srcc, dstc = _pad_edges(edge_index_cites)
    srcb, dstb = _pad_edges(edge_index_written_by)
    ident = jnp.eye(D, dtype=jnp.float32)

    S_w, dgw = _seg_deg(embed_author, srcw, dstw, ident)
    S_c, dgc = _seg_deg(feat_paper, srcc, dstc, ident)
    S_b, dgb = _seg_deg(feat_paper, srcb, dstb, ident)
    invw, invc, invb = _inv_deg(dgw), _inv_deg(dgc), _inv_deg(dgb)

    h1p = _dense([S_w, S_c], [invw, invc], feat_paper,
                 [W1_writes, W1_cites], L1_paper, True, D)
    h1a = _dense([S_b], [invb], embed_author,
                 [W1_written_by], L1_author, True, D)

    S2w = _seg(h1a, srcw, dstw)
    S2c = _seg(h1p, srcc, dstc)

    return _dense([S2w, S2c], [invw, invc], h1p,
                  [W2_writes, W2_cites], L2_paper, False, OUT)


def kernel(feat_paper, edge_index_writes, edge_index_cites,
           edge_index_written_by, embed_author, W1_writes, W1_cites,
           W1_written_by, L1_paper, L1_author, W2_writes, W2_cites,
           W2_written_by, L2_paper, L2_author):
    return _impl(feat_paper, edge_index_writes, edge_index_cites,
                 edge_index_written_by, embed_author, W1_writes, W1_cites,
                 W1_written_by, L1_paper, L1_author, W2_writes, W2_cites,
                 W2_written_by, L2_paper, L2_author)


# speedup vs baseline: 1.7105x; 1.7105x over previous
import functools

import jax
import jax.numpy as jnp
from jax import lax
from jax.experimental import pallas as pl
from jax.experimental.pallas import tpu as pltpu
from jax.experimental.pallas import tpu_sc as plsc

N = 50000
E = 200000
D = 128
OUT = 64
NC, NS = 2, 16
BR = 6400
ACC_R = BR + 64
DUMMY = BR
NBLK = 8
NPAD = NBLK * BR
PT = 12544
EPAD = NS * PT
CH = 3136
NCHUNK = PT // CH
VPC = CH // 16
BATCH = 16
SELCAP = CH + 64
ZR = ACC_R // NS
ZS = 101
WR = BR // NS
DGR = BR // 128
DGA = 56


def _seg_body(with_deg, *refs):
    if with_deg:
        (table, srcp, dstp, ident, s_out, deg_out, acc, dgacc, dst_c, src_c,
         sel_src, sel_dst, idsb0, idxb0, colb0, rowb0, idsb1, idxb1, colb1,
         rowb1, gbuf0, gbuf1, obuf0, obuf1, zrow,
         gfs0, gfs1, gds0, gds1, sfs0, sfs1, sds0, sds1) = refs
        idsb = (idsb0, idsb1)
        idxb = (idxb0, idxb1)
        colb = (colb0, colb1)
        rowb = (rowb0, rowb1)
        gbuf = (gbuf0, gbuf1)
        obuf = (obuf0, obuf1)
        gfs = (gfs0, gfs1)
        gds = (gds0, gds1)
        sfs = (sfs0, sfs1)
        sds = (sds0, sds1)
    else:
        (table, srcp, dstp, s_out, acc, dst_c, src_c,
         sel_src, sel_dst, idsb0, idxb0, idsb1, idxb1,
         gbuf0, gbuf1, zrow, gfs0, gfs1, sfs0, sfs1) = refs
        idsb = (idsb0, idsb1)
        idxb = (idxb0, idxb1)
        gbuf = (gbuf0, gbuf1)
        gfs = (gfs0, gfs1)
        sfs = (sfs0, sfs1)
    c = lax.axis_index("c")
    s = lax.axis_index("s")
    lanes = lax.iota(jnp.int32, 16)
    zf = jnp.zeros((16,), jnp.float32)

    def fill_zrow(i, carry):
        for k in range(D // 16):
            zrow[i, pl.ds(k * 16, 16)] = zf
        return carry
    lax.fori_loop(0, ZS, fill_zrow, 0)

    ebase = s * PT

    def phase_body(p):
        blk = p * NC + c
        lo = blk * BR

        for k in range(ZR // ZS):
            pltpu.sync_copy(zrow, acc.at[pl.ds(s * ZR + k * ZS, ZS)])
        if with_deg:
            @pl.when(s == 0)
            def _():
                pltpu.sync_copy(zrow.at[pl.ds(0, DGA)], dgacc)
        plsc.subcore_barrier()

        def chunk_body(ci, carry):
            cpd = pltpu.async_copy(dstp.at[pl.ds(ebase + ci * CH, CH)],
                                   dst_c, gfs[0])
            cps = pltpu.async_copy(srcp.at[pl.ds(ebase + ci * CH, CH)],
                                   src_c, gfs[1])
            cpd.wait()
            cps.wait()

            def vbody(j, cur):
                dv = dst_c[pl.ds(j * 16, 16)]
                sv = src_c[pl.ds(j * 16, 16)]
                ld = dv - lo
                inbi = (1 - lax.shift_right_logical(ld, 31)) * (
                    1 - lax.shift_right_logical(BR - 1 - ld, 31))
                outi = 1 - inbi
                pos = plsc.cumsum(inbi)
                tgt = inbi * (cur + pos - 1) + outi * (SELCAP - 16 + lanes)
                plsc.store_scatter(sel_src, [tgt], sv)
                plsc.store_scatter(sel_dst, [tgt], ld * inbi + DUMMY * outi)
                return cur + pos[15]
            nsel = lax.fori_loop(0, VPC, vbody, jnp.int32(0))

            tb = (nsel // 16) * 16
            for k in range(BATCH // 16 + 1):
                off = tb + k * 16
                m = (off + lanes) >= nsel
                olds = sel_src[pl.ds(off, 16)]
                oldd = sel_dst[pl.ds(off, 16)]
                sel_src[pl.ds(off, 16)] = jnp.where(m, 0, olds)
                sel_dst[pl.ds(off, 16)] = jnp.where(m, DUMMY, oldd)

            nb = (nsel + BATCH - 1) // BATCH

            def pbody(q, carry2):
                for io in range(2):
                    i = q * 2 + io
                    st = io
                    ot = 1 - io

                    @pl.when((i >= 2) & (i <= nb))
                    def _(st=st, i=i):
                        pltpu.make_async_copy(gbuf[st], acc.at[idxb[st]],
                                              sfs[st]).wait()
                        if with_deg:
                            pltpu.make_async_copy(obuf[st],
                                                  dgacc.at[rowb[st]],
                                                  sds[st]).wait()

                    @pl.when(i < nb)
                    def _(st=st, i=i):
                        for k in range(BATCH // 16):
                            sv_ = sel_src[pl.ds(i * BATCH + k * 16, 16)]
                            dv_ = sel_dst[pl.ds(i * BATCH + k * 16, 16)]
                            idsb[st][pl.ds(k * 16, 16)] = sv_
                            idxb[st][pl.ds(k * 16, 16)] = dv_
                            if with_deg:
                                colb[st][pl.ds(k * 16, 16)] = dv_ & 127
                                rowb[st][pl.ds(k * 16, 16)] = (
                                    lax.shift_right_logical(dv_, 7))
                        pltpu.async_copy(table.at[idsb[st]], gbuf[st],
                                         gfs[st])
                        if with_deg:
                            pltpu.async_copy(ident.at[colb[st]], obuf[st],
                                             gds[st])

                    @pl.when((i >= 1) & (i <= nb))
                    def _(ot=ot, i=i):
                        pltpu.make_async_copy(table.at[idsb[ot]], gbuf[ot],
                                              gfs[ot]).wait()
                        pltpu.async_copy(gbuf[ot], acc.at[idxb[ot]],
                                         sfs[ot], add=True)
                        if with_deg:
                            pltpu.make_async_copy(ident.at[colb[ot]],
                                                  obuf[ot], gds[ot]).wait()
                            pltpu.async_copy(obuf[ot], dgacc.at[rowb[ot]],
                                             sds[ot], add=True)
                return carry2
            lax.fori_loop(0, (nb + 2) // 2, pbody, 0)

            for par in range(2):
                @pl.when((nb >= 1) & (nb % 2 == (1 - par)))
                def _(par=par):
                    pltpu.make_async_copy(gbuf[par], acc.at[idxb[par]],
                                          sfs[par]).wait()
                    if with_deg:
                        pltpu.make_async_copy(obuf[par],
                                              dgacc.at[rowb[par]],
                                              sds[par]).wait()
            return carry
        lax.fori_loop(0, NCHUNK, chunk_body, 0)
        plsc.subcore_barrier()

        pltpu.sync_copy(acc.at[pl.ds(s * WR, WR)],
                        s_out.at[pl.ds(blk * BR + s * WR, WR)])
        if with_deg:
            @pl.when(s == 0)
            def _():
                pltpu.sync_copy(dgacc, deg_out.at[pl.ds(blk * DGA, DGA)])
        plsc.subcore_barrier()

    for p in range(NBLK // NC):
        phase_body(p)


@functools.lru_cache(maxsize=None)
def _make_seg(with_deg):
    out_type = [jax.ShapeDtypeStruct((NPAD, D), jnp.float32)]
    if with_deg:
        out_type.append(jax.ShapeDtypeStruct((NBLK * DGA, D), jnp.float32))
    scratch = [pltpu.VMEM_SHARED((ACC_R, D), jnp.float32)]
    if with_deg:
        scratch.append(pltpu.VMEM_SHARED((DGA, D), jnp.float32))
    scratch += [
        pltpu.VMEM((CH,), jnp.int32),
        pltpu.VMEM((CH,), jnp.int32),
        pltpu.VMEM((SELCAP,), jnp.int32),
        pltpu.VMEM((SELCAP,), jnp.int32),
    ]
    nidx = 4 if with_deg else 2
    for _ in range(2 * nidx):
        scratch.append(pltpu.VMEM((BATCH,), jnp.int32))
    scratch += [
        pltpu.VMEM((BATCH, D), jnp.float32),
        pltpu.VMEM((BATCH, D), jnp.float32),
    ]
    if with_deg:
        scratch += [
            pltpu.VMEM((BATCH, D), jnp.float32),
            pltpu.VMEM((BATCH, D), jnp.float32),
        ]
    scratch.append(pltpu.VMEM((ZS, D), jnp.float32))
    nsem = 8 if with_deg else 4
    for _ in range(nsem):
        scratch.append(pltpu.SemaphoreType.DMA)
    mesh = plsc.VectorSubcoreMesh(core_axis_name="c", subcore_axis_name="s",
                                  num_cores=NC, num_subcores=NS)
    return pl.kernel(functools.partial(_seg_body, with_deg),
                     out_type=tuple(out_type) if with_deg else out_type[0],
                     mesh=mesh,
                     scratch_types=scratch,
                     compiler_params=pltpu.CompilerParams(
                         needs_layout_passes=False),
                     name="seg_sum_deg" if with_deg else "seg_sum")


def _seg_deg(table, src, dst, ident):
    return _make_seg(True)(table, src, dst, ident)


def _seg(table, src, dst):
    return _make_seg(False)(table, src, dst)


def _dense(S_list, inv_list, X, W_list, Lw, relu, out_dim):
    nrel = len(S_list)
    BLK = 1000

    def body(*refs):
        Ss = refs[:nrel]
        invs = refs[nrel:2 * nrel]
        Xr = refs[2 * nrel]
        Ws = refs[2 * nrel + 1:3 * nrel + 1]
        Lr = refs[3 * nrel + 1]
        o = refs[-1]
        acc = jnp.dot(Xr[...], Lr[...], preferred_element_type=jnp.float32)
        for Sr, ir, Wr in zip(Ss, invs, Ws):
            acc = acc + jnp.dot(Sr[...], Wr[...],
                                preferred_element_type=jnp.float32) * ir[:, 0:1]
        o[...] = jnp.maximum(acc, 0.0) if relu else acc

    in_specs = (
        [pl.BlockSpec((BLK, D), lambda i: (i, 0))] * nrel
        + [pl.BlockSpec((BLK, 8), lambda i: (i, 0))] * nrel
        + [pl.BlockSpec((BLK, D), lambda i: (i, 0))]
        + [pl.BlockSpec((D, out_dim), lambda i: (0, 0))] * (nrel + 1)
    )
    return pl.pallas_call(
        body,
        grid=(N // BLK,),
        in_specs=in_specs,
        out_specs=pl.BlockSpec((BLK, out_dim), lambda i: (i, 0)),
        out_shape=jax.ShapeDtypeStruct((N, out_dim), jnp.float32),
    )(*S_list, *inv_list, X, *W_list, Lw)


def _pad_edges(ei):
    src = jnp.pad(ei[0], (0, EPAD - E))
    dst = jnp.pad(ei[1], (0, EPAD - E), constant_values=-1)
    return src, dst


def _inv_deg(deg_out):
    deg = deg_out.reshape(NBLK, DGA, D)[:, :DGR, :].reshape(NBLK * BR)[:N]
    inv = 1.0 / jnp.maximum(deg, 1.0)
    return jnp.broadcast_to(inv[:, None], (N, 8))


@jax.jit
def _impl(feat_paper, edge_index_writes, edge_index_cites,
          edge_index_written_by, embed_author, W1_writes, W1_cites,
          W1_written_by, L1_paper, L1_author, W2_writes, W2_cites,
          W2_written_by, L2_paper, L2_author):
    srcw, dstw = _pad_edges(edge_index_writes)
    srcc, dstc = _pad_edges(edge_index_cites)
    srcb, dstb = _pad_edges(edge_index_written_by)
    ident = jnp.eye(D, dtype=jnp.float32)

    S_w, dgw = _seg_deg(embed_author, srcw, dstw, ident)
    S_c, dgc = _seg_deg(feat_paper, srcc, dstc, ident)
    S_b, dgb = _seg_deg(feat_paper, srcb, dstb, ident)
    invw, invc, invb = _inv_deg(dgw), _inv_deg(dgc), _inv_deg(dgb)

    h1p = _dense([S_w, S_c], [invw, invc], feat_paper,
                 [W1_writes, W1_cites], L1_paper, True, D)
    h1a = _dense([S_b], [invb], embed_author,
                 [W1_written_by], L1_author, True, D)

    S2w = _seg(h1a, srcw, dstw)
    S2c = _seg(h1p, srcc, dstc)

    return _dense([S2w, S2c], [invw, invc], h1p,
                  [W2_writes, W2_cites], L2_paper, False, OUT)


def kernel(feat_paper, edge_index_writes, edge_index_cites,
           edge_index_written_by, embed_author, W1_writes, W1_cites,
           W1_written_by, L1_paper, L1_author, W2_writes, W2_cites,
           W2_written_by, L2_paper, L2_author):
    return _impl(feat_paper, edge_index_writes, edge_index_cites,
                 edge_index_written_by, embed_author, W1_writes, W1_cites,
                 W1_written_by, L1_paper, L1_author, W2_writes, W2_cites,
                 W2_written_by, L2_paper, L2_author)


# speedup vs baseline: 1.9116x; 1.1176x over previous
import functools

import jax
import jax.numpy as jnp
from jax import lax
from jax.experimental import pallas as pl
from jax.experimental.pallas import tpu as pltpu
from jax.experimental.pallas import tpu_sc as plsc

N = 50000
E = 200000
D = 128
OUT = 64
NC, NS = 2, 16
BR = 6400
ACC_R = BR + 64
DUMMY = BR
NBLK = 8
NPAD = NBLK * BR
PT = 12544
EPAD = NS * PT
CH = 6272
NCHUNK = PT // CH
VPC = CH // 16
BATCH = 16
SELCAP = CH + 64
ZR = ACC_R // NS
ZS = 101
WR = BR // NS
DGR = BR // 128
DGA = 56


def _seg_body(with_deg, *refs):
    if with_deg:
        (table, srcp, dstp, ident, s_out, deg_out, acc, dgacc, dst_c, src_c,
         sel_src, sel_dst, idsb0, idxb0, colb0, rowb0, idsb1, idxb1, colb1,
         rowb1, gbuf0, gbuf1, obuf0, obuf1, zrow,
         gfs0, gfs1, gds0, gds1, sfs0, sfs1, sds0, sds1) = refs
        idsb = (idsb0, idsb1)
        idxb = (idxb0, idxb1)
        colb = (colb0, colb1)
        rowb = (rowb0, rowb1)
        gbuf = (gbuf0, gbuf1)
        obuf = (obuf0, obuf1)
        gfs = (gfs0, gfs1)
        gds = (gds0, gds1)
        sfs = (sfs0, sfs1)
        sds = (sds0, sds1)
    else:
        (table, srcp, dstp, s_out, acc, dst_c, src_c,
         sel_src, sel_dst, idsb0, idxb0, idsb1, idxb1,
         gbuf0, gbuf1, zrow, gfs0, gfs1, sfs0, sfs1) = refs
        idsb = (idsb0, idsb1)
        idxb = (idxb0, idxb1)
        gbuf = (gbuf0, gbuf1)
        gfs = (gfs0, gfs1)
        sfs = (sfs0, sfs1)
    c = lax.axis_index("c")
    s = lax.axis_index("s")
    lanes = lax.iota(jnp.int32, 16)
    zf = jnp.zeros((16,), jnp.float32)

    def fill_zrow(i, carry):
        for k in range(D // 16):
            zrow[i, pl.ds(k * 16, 16)] = zf
        return carry
    lax.fori_loop(0, ZS, fill_zrow, 0)

    ebase = s * PT

    def phase_body(p):
        blk = p * NC + c
        lo = blk * BR

        for k in range(ZR // ZS):
            pltpu.sync_copy(zrow, acc.at[pl.ds(s * ZR + k * ZS, ZS)])
        if with_deg:
            @pl.when(s == 0)
            def _():
                pltpu.sync_copy(zrow.at[pl.ds(0, DGA)], dgacc)
        plsc.subcore_barrier()

        def chunk_body(ci, carry):
            cpd = pltpu.async_copy(dstp.at[pl.ds(ebase + ci * CH, CH)],
                                   dst_c, gfs[0])
            cps = pltpu.async_copy(srcp.at[pl.ds(ebase + ci * CH, CH)],
                                   src_c, gfs[1])
            cpd.wait()
            cps.wait()

            def vbody(j, cur):
                dv = dst_c[pl.ds(j * 16, 16)]
                sv = src_c[pl.ds(j * 16, 16)]
                ld = dv - lo
                inbi = (1 - lax.shift_right_logical(ld, 31)) * (
                    1 - lax.shift_right_logical(BR - 1 - ld, 31))
                outi = 1 - inbi
                pos = plsc.cumsum(inbi)
                tgt = inbi * (cur + pos - 1) + outi * (SELCAP - 16 + lanes)
                plsc.store_scatter(sel_src, [tgt], sv)
                plsc.store_scatter(sel_dst, [tgt], ld * inbi + DUMMY * outi)
                return cur + pos[15]
            nsel = lax.fori_loop(0, VPC, vbody, jnp.int32(0))

            tb = (nsel // 16) * 16
            for k in range(BATCH // 16 + 1):
                off = tb + k * 16
                m = (off + lanes) >= nsel
                olds = sel_src[pl.ds(off, 16)]
                oldd = sel_dst[pl.ds(off, 16)]
                sel_src[pl.ds(off, 16)] = jnp.where(m, 0, olds)
                sel_dst[pl.ds(off, 16)] = jnp.where(m, DUMMY, oldd)

            nb = (nsel + BATCH - 1) // BATCH

            def pbody(q, carry2):
                for io in range(2):
                    i = q * 2 + io
                    st = io
                    ot = 1 - io

                    @pl.when((i >= 2) & (i <= nb))
                    def _(st=st, i=i):
                        pltpu.make_async_copy(gbuf[st], acc.at[idxb[st]],
                                              sfs[st]).wait()
                        if with_deg:
                            pltpu.make_async_copy(obuf[st],
                                                  dgacc.at[rowb[st]],
                                                  sds[st]).wait()

                    @pl.when(i < nb)
                    def _(st=st, i=i):
                        for k in range(BATCH // 16):
                            sv_ = sel_src[pl.ds(i * BATCH + k * 16, 16)]
                            dv_ = sel_dst[pl.ds(i * BATCH + k * 16, 16)]
                            idsb[st][pl.ds(k * 16, 16)] = sv_
                            idxb[st][pl.ds(k * 16, 16)] = dv_
                            if with_deg:
                                colb[st][pl.ds(k * 16, 16)] = dv_ & 127
                                rowb[st][pl.ds(k * 16, 16)] = (
                                    lax.shift_right_logical(dv_, 7))
                        pltpu.async_copy(table.at[idsb[st]], gbuf[st],
                                         gfs[st])
                        if with_deg:
                            pltpu.async_copy(ident.at[colb[st]], obuf[st],
                                             gds[st])

                    @pl.when((i >= 1) & (i <= nb))
                    def _(ot=ot, i=i):
                        pltpu.make_async_copy(table.at[idsb[ot]], gbuf[ot],
                                              gfs[ot]).wait()
                        pltpu.async_copy(gbuf[ot], acc.at[idxb[ot]],
                                         sfs[ot], add=True)
                        if with_deg:
                            pltpu.make_async_copy(ident.at[colb[ot]],
                                                  obuf[ot], gds[ot]).wait()
                            pltpu.async_copy(obuf[ot], dgacc.at[rowb[ot]],
                                             sds[ot], add=True)
                return carry2
            lax.fori_loop(0, (nb + 2) // 2, pbody, 0)

            for par in range(2):
                @pl.when((nb >= 1) & (nb % 2 == (1 - par)))
                def _(par=par):
                    pltpu.make_async_copy(gbuf[par], acc.at[idxb[par]],
                                          sfs[par]).wait()
                    if with_deg:
                        pltpu.make_async_copy(obuf[par],
                                              dgacc.at[rowb[par]],
                                              sds[par]).wait()
            return carry
        lax.fori_loop(0, NCHUNK, chunk_body, 0)
        plsc.subcore_barrier()

        pltpu.sync_copy(acc.at[pl.ds(s * WR, WR)],
                        s_out.at[pl.ds(blk * BR + s * WR, WR)])
        if with_deg:
            @pl.when(s == 0)
            def _():
                pltpu.sync_copy(dgacc, deg_out.at[pl.ds(blk * DGA, DGA)])
        plsc.subcore_barrier()

    for p in range(NBLK // NC):
        phase_body(p)


@functools.lru_cache(maxsize=None)
def _make_seg(with_deg):
    out_type = [jax.ShapeDtypeStruct((NPAD, D), jnp.float32)]
    if with_deg:
        out_type.append(jax.ShapeDtypeStruct((NBLK * DGA, D), jnp.float32))
    scratch = [pltpu.VMEM_SHARED((ACC_R, D), jnp.float32)]
    if with_deg:
        scratch.append(pltpu.VMEM_SHARED((DGA, D), jnp.float32))
    scratch += [
        pltpu.VMEM((CH,), jnp.int32),
        pltpu.VMEM((CH,), jnp.int32),
        pltpu.VMEM((SELCAP,), jnp.int32),
        pltpu.VMEM((SELCAP,), jnp.int32),
    ]
    nidx = 4 if with_deg else 2
    for _ in range(2 * nidx):
        scratch.append(pltpu.VMEM((BATCH,), jnp.int32))
    scratch += [
        pltpu.VMEM((BATCH, D), jnp.float32),
        pltpu.VMEM((BATCH, D), jnp.float32),
    ]
    if with_deg:
        scratch += [
            pltpu.VMEM((BATCH, D), jnp.float32),
            pltpu.VMEM((BATCH, D), jnp.float32),
        ]
    scratch.append(pltpu.VMEM((ZS, D), jnp.float32))
    nsem = 8 if with_deg else 4
    for _ in range(nsem):
        scratch.append(pltpu.SemaphoreType.DMA)
    mesh = plsc.VectorSubcoreMesh(core_axis_name="c", subcore_axis_name="s",
                                  num_cores=NC, num_subcores=NS)
    return pl.kernel(functools.partial(_seg_body, with_deg),
                     out_type=tuple(out_type) if with_deg else out_type[0],
                     mesh=mesh,
                     scratch_types=scratch,
                     compiler_params=pltpu.CompilerParams(
                         needs_layout_passes=False),
                     name="seg_sum_deg" if with_deg else "seg_sum")


def _seg_deg(table, src, dst, ident):
    return _make_seg(True)(table, src, dst, ident)


def _seg(table, src, dst):
    return _make_seg(False)(table, src, dst)


def _dense(S_list, inv_list, X, W_list, Lw, relu, out_dim):
    nrel = len(S_list)
    BLK = 1000

    def body(*refs):
        Ss = refs[:nrel]
        invs = refs[nrel:2 * nrel]
        Xr = refs[2 * nrel]
        Ws = refs[2 * nrel + 1:3 * nrel + 1]
        Lr = refs[3 * nrel + 1]
        o = refs[-1]
        acc = jnp.dot(Xr[...], Lr[...], preferred_element_type=jnp.float32)
        for Sr, ir, Wr in zip(Ss, invs, Ws):
            acc = acc + jnp.dot(Sr[...], Wr[...],
                                preferred_element_type=jnp.float32) * ir[:, 0:1]
        o[...] = jnp.maximum(acc, 0.0) if relu else acc

    in_specs = (
        [pl.BlockSpec((BLK, D), lambda i: (i, 0))] * nrel
        + [pl.BlockSpec((BLK, 8), lambda i: (i, 0))] * nrel
        + [pl.BlockSpec((BLK, D), lambda i: (i, 0))]
        + [pl.BlockSpec((D, out_dim), lambda i: (0, 0))] * (nrel + 1)
    )
    return pl.pallas_call(
        body,
        grid=(N // BLK,),
        in_specs=in_specs,
        out_specs=pl.BlockSpec((BLK, out_dim), lambda i: (i, 0)),
        out_shape=jax.ShapeDtypeStruct((N, out_dim), jnp.float32),
    )(*S_list, *inv_list, X, *W_list, Lw)


def _pad_edges(ei):
    src = jnp.pad(ei[0], (0, EPAD - E))
    dst = jnp.pad(ei[1], (0, EPAD - E), constant_values=-1)
    return src, dst


def _inv_deg(deg_out):
    deg = deg_out.reshape(NBLK, DGA, D)[:, :DGR, :].reshape(NBLK * BR)[:N]
    inv = 1.0 / jnp.maximum(deg, 1.0)
    return jnp.broadcast_to(inv[:, None], (N, 8))


@jax.jit
def _impl(feat_paper, edge_index_writes, edge_index_cites,
          edge_index_written_by, embed_author, W1_writes, W1_cites,
          W1_written_by, L1_paper, L1_author, W2_writes, W2_cites,
          W2_written_by, L2_paper, L2_author):
    srcw, dstw = _pad_edges(edge_index_writes)
    srcc, dstc = _pad_edges(edge_index_cites)
    srcb, dstb = _pad_edges(edge_index_written_by)
    ident = jnp.eye(D, dtype=jnp.float32)

    S_w, dgw = _seg_deg(embed_author, srcw, dstw, ident)
    S_c, dgc = _seg_deg(feat_paper, srcc, dstc, ident)
    S_b, dgb = _seg_deg(feat_paper, srcb, dstb, ident)
    invw, invc, invb = _inv_deg(dgw), _inv_deg(dgc), _inv_deg(dgb)

    h1p = _dense([S_w, S_c], [invw, invc], feat_paper,
                 [W1_writes, W1_cites], L1_paper, True, D)
    h1a = _dense([S_b], [invb], embed_author,
                 [W1_written_by], L1_author, True, D)

    S2w = _seg(h1a, srcw, dstw)
    S2c = _seg(h1p, srcc, dstc)

    return _dense([S2w, S2c], [invw, invc], h1p,
                  [W2_writes, W2_cites], L2_paper, False, OUT)


def kernel(feat_paper, edge_index_writes, edge_index_cites,
           edge_index_written_by, embed_author, W1_writes, W1_cites,
           W1_written_by, L1_paper, L1_author, W2_writes, W2_cites,
           W2_written_by, L2_paper, L2_author):
    return _impl(feat_paper, edge_index_writes, edge_index_cites,
                 edge_index_written_by, embed_author, W1_writes, W1_cites,
                 W1_written_by, L1_paper, L1_author, W2_writes, W2_cites,
                 W2_written_by, L2_paper, L2_author)


# speedup vs baseline: 1.9792x; 1.0353x over previous
import functools

import jax
import jax.numpy as jnp
from jax import lax
from jax.experimental import pallas as pl
from jax.experimental.pallas import tpu as pltpu
from jax.experimental.pallas import tpu_sc as plsc

N = 50000
E = 200000
D = 128
OUT = 64
NC, NS = 2, 16
BR = 6400
ACC_R = BR + 64
DUMMY = BR
NBLK = 8
NPAD = NBLK * BR
PT = 12544
EPAD = NS * PT
CH = 6272
NCHUNK = PT // CH
VPC = CH // 16
BATCH = 16
SELCAP = CH + 64
ZR = ACC_R // NS
ZS = 101
WR = BR // NS
DGR = BR // 128
DGA = 56
DP = 4


def _seg_body(with_deg, *refs):
    if with_deg:
        (table, srcp, dstp, ident, s_out, deg_out, acc, dgacc, dst_c, src_c,
         sel_src, sel_dst) = refs[:12]
        k = 12
        idsb = refs[k:k + 4 * DP:4]
        idxb = refs[k + 1:k + 4 * DP:4]
        colb = refs[k + 2:k + 4 * DP:4]
        rowb = refs[k + 3:k + 4 * DP:4]
        k += 4 * DP
        gbuf = refs[k:k + DP]
        obuf = refs[k + DP:k + 2 * DP]
        zrow = refs[k + 2 * DP]
        k += 2 * DP + 1
        gfs = refs[k:k + DP]
        gds = refs[k + DP:k + 2 * DP]
        sfs = refs[k + 2 * DP:k + 3 * DP]
        sds = refs[k + 3 * DP:k + 4 * DP]
    else:
        (table, srcp, dstp, s_out, acc, dst_c, src_c,
         sel_src, sel_dst) = refs[:9]
        k = 9
        idsb = refs[k:k + 2 * DP:2]
        idxb = refs[k + 1:k + 2 * DP:2]
        k += 2 * DP
        gbuf = refs[k:k + DP]
        zrow = refs[k + DP]
        k += DP + 1
        gfs = refs[k:k + DP]
        sfs = refs[k + DP:k + 2 * DP]
    c = lax.axis_index("c")
    s = lax.axis_index("s")
    lanes = lax.iota(jnp.int32, 16)
    zf = jnp.zeros((16,), jnp.float32)

    def fill_zrow(i, carry):
        for k in range(D // 16):
            zrow[i, pl.ds(k * 16, 16)] = zf
        return carry
    lax.fori_loop(0, ZS, fill_zrow, 0)

    ebase = s * PT

    def phase_body(p):
        blk = p * NC + c
        lo = blk * BR

        for k in range(ZR // ZS):
            pltpu.sync_copy(zrow, acc.at[pl.ds(s * ZR + k * ZS, ZS)])
        if with_deg:
            @pl.when(s == 0)
            def _():
                pltpu.sync_copy(zrow.at[pl.ds(0, DGA)], dgacc)
        plsc.subcore_barrier()

        def chunk_body(ci, carry):
            cpd = pltpu.async_copy(dstp.at[pl.ds(ebase + ci * CH, CH)],
                                   dst_c, gfs[0])
            cps = pltpu.async_copy(srcp.at[pl.ds(ebase + ci * CH, CH)],
                                   src_c, gfs[1])
            cpd.wait()
            cps.wait()

            def vbody(j, cur):
                dv = dst_c[pl.ds(j * 16, 16)]
                sv = src_c[pl.ds(j * 16, 16)]
                ld = dv - lo
                inbi = (1 - lax.shift_right_logical(ld, 31)) * (
                    1 - lax.shift_right_logical(BR - 1 - ld, 31))
                outi = 1 - inbi
                pos = plsc.cumsum(inbi)
                tgt = inbi * (cur + pos - 1) + outi * (SELCAP - 16 + lanes)
                plsc.store_scatter(sel_src, [tgt], sv)
                plsc.store_scatter(sel_dst, [tgt], ld * inbi + DUMMY * outi)
                return cur + pos[15]
            nsel = lax.fori_loop(0, VPC, vbody, jnp.int32(0))

            tb = (nsel // 16) * 16
            for k in range(BATCH // 16 + 1):
                off = tb + k * 16
                m = (off + lanes) >= nsel
                olds = sel_src[pl.ds(off, 16)]
                oldd = sel_dst[pl.ds(off, 16)]
                sel_src[pl.ds(off, 16)] = jnp.where(m, 0, olds)
                sel_dst[pl.ds(off, 16)] = jnp.where(m, DUMMY, oldd)

            nb = (nsel + BATCH - 1) // BATCH

            def pbody(q, carry2):
                for io in range(DP):
                    i = q * DP + io
                    st = io
                    ot = (io + DP - 1) % DP

                    @pl.when((i >= DP) & (i <= nb))
                    def _(st=st, i=i):
                        pltpu.make_async_copy(gbuf[st], acc.at[idxb[st]],
                                              sfs[st]).wait()
                        if with_deg:
                            pltpu.make_async_copy(obuf[st],
                                                  dgacc.at[rowb[st]],
                                                  sds[st]).wait()

                    @pl.when(i < nb)
                    def _(st=st, i=i):
                        for k in range(BATCH // 16):
                            sv_ = sel_src[pl.ds(i * BATCH + k * 16, 16)]
                            dv_ = sel_dst[pl.ds(i * BATCH + k * 16, 16)]
                            idsb[st][pl.ds(k * 16, 16)] = sv_
                            idxb[st][pl.ds(k * 16, 16)] = dv_
                            if with_deg:
                                colb[st][pl.ds(k * 16, 16)] = dv_ & 127
                                rowb[st][pl.ds(k * 16, 16)] = (
                                    lax.shift_right_logical(dv_, 7))
                        pltpu.async_copy(table.at[idsb[st]], gbuf[st],
                                         gfs[st])
                        if with_deg:
                            pltpu.async_copy(ident.at[colb[st]], obuf[st],
                                             gds[st])

                    @pl.when((i >= 1) & (i <= nb))
                    def _(ot=ot, i=i):
                        pltpu.make_async_copy(table.at[idsb[ot]], gbuf[ot],
                                              gfs[ot]).wait()
                        pltpu.async_copy(gbuf[ot], acc.at[idxb[ot]],
                                         sfs[ot], add=True)
                        if with_deg:
                            pltpu.make_async_copy(ident.at[colb[ot]],
                                                  obuf[ot], gds[ot]).wait()
                            pltpu.async_copy(obuf[ot], dgacc.at[rowb[ot]],
                                             sds[ot], add=True)
                return carry2
            lax.fori_loop(0, (nb + DP) // DP, pbody, 0)

            for m in range(DP):
                for t in range(1, DP):
                    @pl.when((nb % DP == m) & (nb >= t))
                    def _(st=(m - t) % DP):
                        pltpu.make_async_copy(gbuf[st], acc.at[idxb[st]],
                                              sfs[st]).wait()
                        if with_deg:
                            pltpu.make_async_copy(obuf[st],
                                                  dgacc.at[rowb[st]],
                                                  sds[st]).wait()
            return carry
        lax.fori_loop(0, NCHUNK, chunk_body, 0)
        plsc.subcore_barrier()

        pltpu.sync_copy(acc.at[pl.ds(s * WR, WR)],
                        s_out.at[pl.ds(blk * BR + s * WR, WR)])
        if with_deg:
            @pl.when(s == 0)
            def _():
                pltpu.sync_copy(dgacc, deg_out.at[pl.ds(blk * DGA, DGA)])
        plsc.subcore_barrier()

    for p in range(NBLK // NC):
        phase_body(p)


@functools.lru_cache(maxsize=None)
def _make_seg(with_deg):
    out_type = [jax.ShapeDtypeStruct((NPAD, D), jnp.float32)]
    if with_deg:
        out_type.append(jax.ShapeDtypeStruct((NBLK * DGA, D), jnp.float32))
    scratch = [pltpu.VMEM_SHARED((ACC_R, D), jnp.float32)]
    if with_deg:
        scratch.append(pltpu.VMEM_SHARED((DGA, D), jnp.float32))
    scratch += [
        pltpu.VMEM((CH,), jnp.int32),
        pltpu.VMEM((CH,), jnp.int32),
        pltpu.VMEM((SELCAP,), jnp.int32),
        pltpu.VMEM((SELCAP,), jnp.int32),
    ]
    nidx = 4 if with_deg else 2
    for _ in range(DP * nidx):
        scratch.append(pltpu.VMEM((BATCH,), jnp.int32))
    nbig = 2 * DP if with_deg else DP
    for _ in range(nbig):
        scratch.append(pltpu.VMEM((BATCH, D), jnp.float32))
    scratch.append(pltpu.VMEM((ZS, D), jnp.float32))
    nsem = 4 * DP if with_deg else 2 * DP
    for _ in range(nsem):
        scratch.append(pltpu.SemaphoreType.DMA)
    mesh = plsc.VectorSubcoreMesh(core_axis_name="c", subcore_axis_name="s",
                                  num_cores=NC, num_subcores=NS)
    return pl.kernel(functools.partial(_seg_body, with_deg),
                     out_type=tuple(out_type) if with_deg else out_type[0],
                     mesh=mesh,
                     scratch_types=scratch,
                     compiler_params=pltpu.CompilerParams(
                         needs_layout_passes=False),
                     name="seg_sum_deg" if with_deg else "seg_sum")


def _seg_deg(table, src, dst, ident):
    return _make_seg(True)(table, src, dst, ident)


def _seg(table, src, dst):
    return _make_seg(False)(table, src, dst)


def _dense(S_list, inv_list, X, W_list, Lw, relu, out_dim):
    nrel = len(S_list)
    BLK = 1000

    def body(*refs):
        Ss = refs[:nrel]
        invs = refs[nrel:2 * nrel]
        Xr = refs[2 * nrel]
        Ws = refs[2 * nrel + 1:3 * nrel + 1]
        Lr = refs[3 * nrel + 1]
        o = refs[-1]
        acc = jnp.dot(Xr[...], Lr[...], preferred_element_type=jnp.float32)
        for Sr, ir, Wr in zip(Ss, invs, Ws):
            acc = acc + jnp.dot(Sr[...], Wr[...],
                                preferred_element_type=jnp.float32) * ir[:, 0:1]
        o[...] = jnp.maximum(acc, 0.0) if relu else acc

    in_specs = (
        [pl.BlockSpec((BLK, D), lambda i: (i, 0))] * nrel
        + [pl.BlockSpec((BLK, 8), lambda i: (i, 0))] * nrel
        + [pl.BlockSpec((BLK, D), lambda i: (i, 0))]
        + [pl.BlockSpec((D, out_dim), lambda i: (0, 0))] * (nrel + 1)
    )
    return pl.pallas_call(
        body,
        grid=(N // BLK,),
        in_specs=in_specs,
        out_specs=pl.BlockSpec((BLK, out_dim), lambda i: (i, 0)),
        out_shape=jax.ShapeDtypeStruct((N, out_dim), jnp.float32),
    )(*S_list, *inv_list, X, *W_list, Lw)


def _pad_edges(ei):
    src = jnp.pad(ei[0], (0, EPAD - E))
    dst = jnp.pad(ei[1], (0, EPAD - E), constant_values=-1)
    return src, dst


def _inv_deg(deg_out):
    deg = deg_out.reshape(NBLK, DGA, D)[:, :DGR, :].reshape(NBLK * BR)[:N]
    inv = 1.0 / jnp.maximum(deg, 1.0)
    return jnp.broadcast_to(inv[:, None], (N, 8))


@jax.jit
def _impl(feat_paper, edge_index_writes, edge_index_cites,
          edge_index_written_by, embed_author, W1_writes, W1_cites,
          W1_written_by, L1_paper, L1_author, W2_writes, W2_cites,
          W2_written_by, L2_paper, L2_author):
    srcw, dstw = _pad_edges(edge_index_writes)
    srcc, dstc = _pad_edges(edge_index_cites)
    srcb, dstb = _pad_edges(edge_index_written_by)
    ident = jnp.eye(D, dtype=jnp.float32)

    S_w, dgw = _seg_deg(embed_author, srcw, dstw, ident)
    S_c, dgc = _seg_deg(feat_paper, srcc, dstc, ident)
    S_b, dgb = _seg_deg(feat_paper, srcb, dstb, ident)
    invw, invc, invb = _inv_deg(dgw), _inv_deg(dgc), _inv_deg(dgb)

    h1p = _dense([S_w, S_c], [invw, invc], feat_paper,
                 [W1_writes, W1_cites], L1_paper, True, D)
    h1a = _dense([S_b], [invb], embed_author,
                 [W1_written_by], L1_author, True, D)

    S2w = _seg(h1a, srcw, dstw)
    S2c = _seg(h1p, srcc, dstc)

    return _dense([S2w, S2c], [invw, invc], h1p,
                  [W2_writes, W2_cites], L2_paper, False, OUT)


def kernel(feat_paper, edge_index_writes, edge_index_cites,
           edge_index_written_by, embed_author, W1_writes, W1_cites,
           W1_written_by, L1_paper, L1_author, W2_writes, W2_cites,
           W2_written_by, L2_paper, L2_author):
    return _impl(feat_paper, edge_index_writes, edge_index_cites,
                 edge_index_written_by, embed_author, W1_writes, W1_cites,
                 W1_written_by, L1_paper, L1_author, W2_writes, W2_cites,
                 W2_written_by, L2_paper, L2_author)
